# Initial kernel scaffold; baseline (speedup 1.0000x reference)
import jax
import jax.numpy as jnp
from jax.experimental import pallas as pl


def kernel(pair_i, pair_j, pair_diff, atom_types, sig, eps):
    raise NotImplementedError("write your pallas kernel here")


# speedup vs baseline: 27.7704x; 27.7704x over previous
import functools

import jax
import jax.numpy as jnp
from jax import lax
from jax.experimental import pallas as pl
from jax.experimental.pallas import tpu as pltpu
from jax.experimental.pallas import tpu_sc as plsc

N_NODES = 100000
N_EDGES = 6400000
N_TYPES = 8

NC = 2
NS = 16
L = 16
NW = NC * NS

CHUNK = 4000


def _lj_body(pair_i_hbm, pair_j_hbm, diff_hbm, types_hbm, sig_hbm, eps_hbm,
             out_hbm,
             types_v, s2_v, e4_v, sig_raw_v, eps_raw_v, pi_v, pj_v, diff_v,
             out_v):
    wid = lax.axis_index("s") * NC + lax.axis_index("c")
    edges_per_tile = N_EDGES // NW
    n_chunks = edges_per_tile // CHUNK
    tile_base = wid * edges_per_tile

    pltpu.sync_copy(types_hbm, types_v)
    pltpu.sync_copy(sig_hbm, sig_raw_v)
    pltpu.sync_copy(eps_hbm, eps_raw_v)

    for t in range(64 // L):
        sl = pl.ds(t * L, L)
        sv = sig_raw_v[sl]
        ev = eps_raw_v[sl]
        s2_v[sl] = sv * sv
        e4_v[sl] = 4.0 * ev

    tri = lax.iota(jnp.int32, L) * 3

    def group_body(g, carry):
        off = g * L
        pi = pi_v[pl.ds(off, L)]
        pj = pj_v[pl.ds(off, L)]
        ti = plsc.load_gather(types_v, [pi])
        tj = plsc.load_gather(types_v, [pj])
        idx = ti * N_TYPES + tj
        s2 = plsc.load_gather(s2_v, [idx])
        e4 = plsc.load_gather(e4_v, [idx])
        dbase = tri + off * 3
        x = plsc.load_gather(diff_v, [dbase])
        y = plsc.load_gather(diff_v, [dbase + 1])
        z = plsc.load_gather(diff_v, [dbase + 2])
        d2 = x * x + y * y + z * z
        q = s2 / d2
        p6 = q * q * q
        out_v[pl.ds(off, L)] = e4 * (p6 * p6 - p6)
        return carry

    def chunk_body(k, carry):
        base = tile_base + k * CHUNK
        pltpu.sync_copy(pair_i_hbm.at[pl.ds(base, CHUNK)], pi_v)
        pltpu.sync_copy(pair_j_hbm.at[pl.ds(base, CHUNK)], pj_v)
        pltpu.sync_copy(diff_hbm.at[pl.ds(base * 3, CHUNK * 3)], diff_v)
        lax.fori_loop(0, CHUNK // L, group_body, 0)
        pltpu.sync_copy(out_v, out_hbm.at[pl.ds(base, CHUNK)])
        return carry

    lax.fori_loop(0, n_chunks, chunk_body, 0)


@jax.jit
def _lj_sc(pair_i, pair_j, pair_diff, atom_types, sig_flat, eps_flat):
    mesh = plsc.VectorSubcoreMesh(core_axis_name="c", subcore_axis_name="s")
    return pl.kernel(
        _lj_body,
        out_type=jax.ShapeDtypeStruct((N_EDGES,), jnp.float32),
        mesh=mesh,
        compiler_params=pltpu.CompilerParams(needs_layout_passes=False),
        scratch_types=[
            pltpu.VMEM((N_NODES,), jnp.int32),
            pltpu.VMEM((64,), jnp.float32),
            pltpu.VMEM((64,), jnp.float32),
            pltpu.VMEM((64,), jnp.float32),
            pltpu.VMEM((64,), jnp.float32),
            pltpu.VMEM((CHUNK,), jnp.int32),
            pltpu.VMEM((CHUNK,), jnp.int32),
            pltpu.VMEM((CHUNK * 3,), jnp.float32),
            pltpu.VMEM((CHUNK,), jnp.float32),
        ],
    )(pair_i, pair_j, pair_diff, atom_types, sig_flat, eps_flat)


def kernel(pair_i, pair_j, pair_diff, atom_types, sig, eps):
    return _lj_sc(pair_i, pair_j, pair_diff.reshape(-1), atom_types,
                  sig.reshape(64), eps.reshape(64))


# speedup vs baseline: 28.8980x; 1.0406x over previous
import jax
import jax.numpy as jnp
from jax import lax
from jax.experimental import pallas as pl
from jax.experimental.pallas import tpu as pltpu
from jax.experimental.pallas import tpu_sc as plsc

N_NODES = 100000
N_EDGES = 6400000
N_TYPES = 8

NC = 2
NS = 16
L = 16
NW = NC * NS

CHUNK = 2000
UNROLL = 5


def _lj_body(pair_i_hbm, pair_j_hbm, diff_hbm, types_hbm, sig_hbm, eps_hbm,
             out_hbm,
             types_v, s2_v, e4_v, sig_raw_v, eps_raw_v,
             pi0, pj0, dif0, o0, pi1, pj1, dif1, o1,
             sin0, sin1, sout0, sout1):
    wid = lax.axis_index("s") * NC + lax.axis_index("c")
    edges_per_tile = N_EDGES // NW
    n_chunks = edges_per_tile // CHUNK
    tile_base = wid * edges_per_tile

    bufs = ((pi0, pj0, dif0, o0, sin0, sout0),
            (pi1, pj1, dif1, o1, sin1, sout1))

    def in_copies(k, b):
        base = tile_base + k * CHUNK
        pi_v, pj_v, diff_v, _, sem, _ = bufs[b]
        return (
            pltpu.make_async_copy(
                pair_i_hbm.at[pl.ds(base, CHUNK)], pi_v, sem),
            pltpu.make_async_copy(
                pair_j_hbm.at[pl.ds(base, CHUNK)], pj_v, sem),
            pltpu.make_async_copy(
                diff_hbm.at[pl.ds(base * 3, CHUNK * 3)], diff_v, sem),
        )

    def start_in(k, b):
        for c in in_copies(k, b):
            c.start()

    def wait_in(k, b):
        for c in in_copies(k, b):
            c.wait()

    def out_copy(k, b):
        base = tile_base + k * CHUNK
        return pltpu.make_async_copy(
            bufs[b][3], out_hbm.at[pl.ds(base, CHUNK)], bufs[b][5])

    start_in(0, 0)
    pltpu.sync_copy(types_hbm, types_v)
    pltpu.sync_copy(sig_hbm, sig_raw_v)
    pltpu.sync_copy(eps_hbm, eps_raw_v)

    for t in range(64 // L):
        sl = pl.ds(t * L, L)
        sv = sig_raw_v[sl]
        ev = eps_raw_v[sl]
        s2_v[sl] = sv * sv
        e4_v[sl] = 4.0 * ev

    tri = lax.iota(jnp.int32, L) * 3

    def compute(b):
        pi_v, pj_v, diff_v, out_v = bufs[b][:4]

        @plsc.parallel_loop(0, CHUNK // L, unroll=UNROLL)
        def group_body(g):
            off = g * L
            pi = pi_v[pl.ds(off, L)]
            pj = pj_v[pl.ds(off, L)]
            ti = plsc.load_gather(types_v, [pi])
            tj = plsc.load_gather(types_v, [pj])
            idx = ti * N_TYPES + tj
            s2 = plsc.load_gather(s2_v, [idx])
            e4 = plsc.load_gather(e4_v, [idx])
            dbase = tri + off * 3
            x = plsc.load_gather(diff_v, [dbase])
            y = plsc.load_gather(diff_v, [dbase + 1])
            z = plsc.load_gather(diff_v, [dbase + 2])
            d2 = x * x + y * y + z * z
            q = s2 / d2
            p6 = q * q * q
            out_v[pl.ds(off, L)] = e4 * (p6 * p6 - p6)

    @pl.loop(0, n_chunks, step=2)
    def pair_body(k2):
        for b in (0, 1):
            kk = k2 + b

            @pl.when(kk + 1 < n_chunks)
            def _():
                start_in(kk + 1, 1 - b)

            wait_in(kk, b)

            @pl.when(kk >= 2)
            def _():
                out_copy(kk - 2, b).wait()

            compute(b)
            out_copy(kk, b).start()

    out_copy(n_chunks - 2, 0).wait()
    out_copy(n_chunks - 1, 1).wait()


@jax.jit
def _lj_sc(pair_i, pair_j, pair_diff, atom_types, sig_flat, eps_flat):
    mesh = plsc.VectorSubcoreMesh(core_axis_name="c", subcore_axis_name="s")
    return pl.kernel(
        _lj_body,
        out_type=jax.ShapeDtypeStruct((N_EDGES,), jnp.float32),
        mesh=mesh,
        compiler_params=pltpu.CompilerParams(needs_layout_passes=False),
        scratch_types=[
            pltpu.VMEM((N_NODES,), jnp.int32),
            pltpu.VMEM((64,), jnp.float32),
            pltpu.VMEM((64,), jnp.float32),
            pltpu.VMEM((64,), jnp.float32),
            pltpu.VMEM((64,), jnp.float32),
            pltpu.VMEM((CHUNK,), jnp.int32),
            pltpu.VMEM((CHUNK,), jnp.int32),
            pltpu.VMEM((CHUNK * 3,), jnp.float32),
            pltpu.VMEM((CHUNK,), jnp.float32),
            pltpu.VMEM((CHUNK,), jnp.int32),
            pltpu.VMEM((CHUNK,), jnp.int32),
            pltpu.VMEM((CHUNK * 3,), jnp.float32),
            pltpu.VMEM((CHUNK,), jnp.float32),
            pltpu.SemaphoreType.DMA,
            pltpu.SemaphoreType.DMA,
            pltpu.SemaphoreType.DMA,
            pltpu.SemaphoreType.DMA,
        ],
    )(pair_i, pair_j, pair_diff, atom_types, sig_flat, eps_flat)


def kernel(pair_i, pair_j, pair_diff, atom_types, sig, eps):
    return _lj_sc(pair_i, pair_j, pair_diff.reshape(-1), atom_types,
                  sig.reshape(64), eps.reshape(64))


# speedup vs baseline: 57.6268x; 1.9941x over previous
import jax
import jax.numpy as jnp
from jax import lax
from jax.experimental import pallas as pl
from jax.experimental.pallas import tpu as pltpu
from jax.experimental.pallas import tpu_sc as plsc

N_NODES = 100000
N_EDGES = 6400000
N_TYPES = 8

NC = 2
NS = 16
L = 16
NW = NC * NS

CHUNK = 2000
UNROLL = 5

D2_BLOCK = 10240


def _d2_body(diff_ref, out_ref):
    d = diff_ref[...]
    out_ref[...] = jnp.sum(d * d, axis=1)


@jax.jit
def _d2_tc(pair_diff):
    grid = N_EDGES // D2_BLOCK
    return pl.pallas_call(
        _d2_body,
        grid=(grid,),
        in_specs=[pl.BlockSpec((D2_BLOCK, 3), lambda i: (i, 0))],
        out_specs=pl.BlockSpec((D2_BLOCK,), lambda i: (i,)),
        out_shape=jax.ShapeDtypeStruct((N_EDGES,), jnp.float32),
    )(pair_diff)


def _lj_body(pair_i_hbm, pair_j_hbm, d2_hbm, types_hbm, sig_hbm, eps_hbm,
             out_hbm,
             types_v, s2_v, e4_v, sig_raw_v, eps_raw_v,
             pi0, pj0, dd0, o0, pi1, pj1, dd1, o1,
             sin0, sin1, sout0, sout1):
    wid = lax.axis_index("s") * NC + lax.axis_index("c")
    edges_per_tile = N_EDGES // NW
    n_chunks = edges_per_tile // CHUNK
    tile_base = wid * edges_per_tile

    bufs = ((pi0, pj0, dd0, o0, sin0, sout0),
            (pi1, pj1, dd1, o1, sin1, sout1))

    def in_copies(k, b):
        base = tile_base + k * CHUNK
        pi_v, pj_v, dd_v, _, sem, _ = bufs[b]
        return (
            pltpu.make_async_copy(
                pair_i_hbm.at[pl.ds(base, CHUNK)], pi_v, sem),
            pltpu.make_async_copy(
                pair_j_hbm.at[pl.ds(base, CHUNK)], pj_v, sem),
            pltpu.make_async_copy(
                d2_hbm.at[pl.ds(base, CHUNK)], dd_v, sem),
        )

    def start_in(k, b):
        for c in in_copies(k, b):
            c.start()

    def wait_in(k, b):
        for c in in_copies(k, b):
            c.wait()

    def out_copy(k, b):
        base = tile_base + k * CHUNK
        return pltpu.make_async_copy(
            bufs[b][3], out_hbm.at[pl.ds(base, CHUNK)], bufs[b][5])

    start_in(0, 0)
    pltpu.sync_copy(types_hbm, types_v)
    pltpu.sync_copy(sig_hbm, sig_raw_v)
    pltpu.sync_copy(eps_hbm, eps_raw_v)

    for t in range(64 // L):
        sl = pl.ds(t * L, L)
        sv = sig_raw_v[sl]
        ev = eps_raw_v[sl]
        s2_v[sl] = sv * sv
        e4_v[sl] = 4.0 * ev

    def compute(b):
        pi_v, pj_v, dd_v, out_v = bufs[b][:4]

        @plsc.parallel_loop(0, CHUNK // L, unroll=UNROLL)
        def group_body(g):
            off = g * L
            pi = pi_v[pl.ds(off, L)]
            pj = pj_v[pl.ds(off, L)]
            ti = plsc.load_gather(types_v, [pi])
            tj = plsc.load_gather(types_v, [pj])
            idx = ti * N_TYPES + tj
            s2 = plsc.load_gather(s2_v, [idx])
            e4 = plsc.load_gather(e4_v, [idx])
            d2 = dd_v[pl.ds(off, L)]
            q = s2 / d2
            p6 = q * q * q
            out_v[pl.ds(off, L)] = e4 * (p6 * p6 - p6)

    @pl.loop(0, n_chunks, step=2)
    def pair_body(k2):
        for b in (0, 1):
            kk = k2 + b

            @pl.when(kk + 1 < n_chunks)
            def _():
                start_in(kk + 1, 1 - b)

            wait_in(kk, b)

            @pl.when(kk >= 2)
            def _():
                out_copy(kk - 2, b).wait()

            compute(b)
            out_copy(kk, b).start()

    out_copy(n_chunks - 2, 0).wait()
    out_copy(n_chunks - 1, 1).wait()


@jax.jit
def _lj_sc(pair_i, pair_j, d2, atom_types, sig_flat, eps_flat):
    mesh = plsc.VectorSubcoreMesh(core_axis_name="c", subcore_axis_name="s")
    return pl.kernel(
        _lj_body,
        out_type=jax.ShapeDtypeStruct((N_EDGES,), jnp.float32),
        mesh=mesh,
        compiler_params=pltpu.CompilerParams(needs_layout_passes=False),
        scratch_types=[
            pltpu.VMEM((N_NODES,), jnp.int32),
            pltpu.VMEM((64,), jnp.float32),
            pltpu.VMEM((64,), jnp.float32),
            pltpu.VMEM((64,), jnp.float32),
            pltpu.VMEM((64,), jnp.float32),
            pltpu.VMEM((CHUNK,), jnp.int32),
            pltpu.VMEM((CHUNK,), jnp.int32),
            pltpu.VMEM((CHUNK,), jnp.float32),
            pltpu.VMEM((CHUNK,), jnp.float32),
            pltpu.VMEM((CHUNK,), jnp.int32),
            pltpu.VMEM((CHUNK,), jnp.int32),
            pltpu.VMEM((CHUNK,), jnp.float32),
            pltpu.VMEM((CHUNK,), jnp.float32),
            pltpu.SemaphoreType.DMA,
            pltpu.SemaphoreType.DMA,
            pltpu.SemaphoreType.DMA,
            pltpu.SemaphoreType.DMA,
        ],
    )(pair_i, pair_j, d2, atom_types, sig_flat, eps_flat)


def kernel(pair_i, pair_j, pair_diff, atom_types, sig, eps):
    d2 = _d2_tc(pair_diff)
    return _lj_sc(pair_i, pair_j, d2, atom_types,
                  sig.reshape(64), eps.reshape(64))


# speedup vs baseline: 97.2229x; 1.6871x over previous
import jax
import jax.numpy as jnp
from jax import lax
from jax.experimental import pallas as pl
from jax.experimental.pallas import tpu as pltpu
from jax.experimental.pallas import tpu_sc as plsc

N_NODES = 100000
N_EDGES = 6400000
N_TYPES = 8

NC = 2
NS = 16
L = 16
NW = NC * NS

CHUNK = 2000
UNROLL = 5

D2_BLOCK = 51200


def _d2_body(diff_ref, out_ref):
    dt = diff_ref[...].T
    out_ref[...] = dt[0] * dt[0] + dt[1] * dt[1] + dt[2] * dt[2]


@jax.jit
def _d2_tc(pair_diff):
    grid = N_EDGES // D2_BLOCK
    return pl.pallas_call(
        _d2_body,
        grid=(grid,),
        in_specs=[pl.BlockSpec((D2_BLOCK, 3), lambda i: (i, 0))],
        out_specs=pl.BlockSpec((D2_BLOCK,), lambda i: (i,)),
        out_shape=jax.ShapeDtypeStruct((N_EDGES,), jnp.float32),
        compiler_params=pltpu.CompilerParams(
            dimension_semantics=("arbitrary",)),
    )(pair_diff)


def _lj_body(pair_i_hbm, pair_j_hbm, d2_hbm, types_hbm, sig_hbm, eps_hbm,
             out_hbm,
             types_v, s2_v, e4_v, sig_raw_v, eps_raw_v,
             pi0, pj0, dd0, o0, pi1, pj1, dd1, o1,
             sin0, sin1, sout0, sout1):
    wid = lax.axis_index("s") * NC + lax.axis_index("c")
    edges_per_tile = N_EDGES // NW
    n_chunks = edges_per_tile // CHUNK
    tile_base = wid * edges_per_tile

    bufs = ((pi0, pj0, dd0, o0, sin0, sout0),
            (pi1, pj1, dd1, o1, sin1, sout1))

    def in_copies(k, b):
        base = tile_base + k * CHUNK
        pi_v, pj_v, dd_v, _, sem, _ = bufs[b]
        return (
            pltpu.make_async_copy(
                pair_i_hbm.at[pl.ds(base, CHUNK)], pi_v, sem),
            pltpu.make_async_copy(
                pair_j_hbm.at[pl.ds(base, CHUNK)], pj_v, sem),
            pltpu.make_async_copy(
                d2_hbm.at[pl.ds(base, CHUNK)], dd_v, sem),
        )

    def start_in(k, b):
        for c in in_copies(k, b):
            c.start()

    def wait_in(k, b):
        for c in in_copies(k, b):
            c.wait()

    def out_copy(k, b):
        base = tile_base + k * CHUNK
        return pltpu.make_async_copy(
            bufs[b][3], out_hbm.at[pl.ds(base, CHUNK)], bufs[b][5])

    start_in(0, 0)
    pltpu.sync_copy(types_hbm, types_v)
    pltpu.sync_copy(sig_hbm, sig_raw_v)
    pltpu.sync_copy(eps_hbm, eps_raw_v)

    for t in range(64 // L):
        sl = pl.ds(t * L, L)
        sv = sig_raw_v[sl]
        ev = eps_raw_v[sl]
        s2_v[sl] = sv * sv
        e4_v[sl] = 4.0 * ev

    def compute(b):
        pi_v, pj_v, dd_v, out_v = bufs[b][:4]

        @plsc.parallel_loop(0, CHUNK // L, unroll=UNROLL)
        def group_body(g):
            off = g * L
            pi = pi_v[pl.ds(off, L)]
            pj = pj_v[pl.ds(off, L)]
            ti = plsc.load_gather(types_v, [pi])
            tj = plsc.load_gather(types_v, [pj])
            idx = ti * N_TYPES + tj
            s2 = plsc.load_gather(s2_v, [idx])
            e4 = plsc.load_gather(e4_v, [idx])
            d2 = dd_v[pl.ds(off, L)]
            q = s2 / d2
            p6 = q * q * q
            out_v[pl.ds(off, L)] = e4 * (p6 * p6 - p6)

    @pl.loop(0, n_chunks, step=2)
    def pair_body(k2):
        for b in (0, 1):
            kk = k2 + b

            @pl.when(kk + 1 < n_chunks)
            def _():
                start_in(kk + 1, 1 - b)

            wait_in(kk, b)

            @pl.when(kk >= 2)
            def _():
                out_copy(kk - 2, b).wait()

            compute(b)
            out_copy(kk, b).start()

    out_copy(n_chunks - 2, 0).wait()
    out_copy(n_chunks - 1, 1).wait()


@jax.jit
def _lj_sc(pair_i, pair_j, d2, atom_types, sig_flat, eps_flat):
    mesh = plsc.VectorSubcoreMesh(core_axis_name="c", subcore_axis_name="s")
    return pl.kernel(
        _lj_body,
        out_type=jax.ShapeDtypeStruct((N_EDGES,), jnp.float32),
        mesh=mesh,
        compiler_params=pltpu.CompilerParams(needs_layout_passes=False),
        scratch_types=[
            pltpu.VMEM((N_NODES,), jnp.int32),
            pltpu.VMEM((64,), jnp.float32),
            pltpu.VMEM((64,), jnp.float32),
            pltpu.VMEM((64,), jnp.float32),
            pltpu.VMEM((64,), jnp.float32),
            pltpu.VMEM((CHUNK,), jnp.int32),
            pltpu.VMEM((CHUNK,), jnp.int32),
            pltpu.VMEM((CHUNK,), jnp.float32),
            pltpu.VMEM((CHUNK,), jnp.float32),
            pltpu.VMEM((CHUNK,), jnp.int32),
            pltpu.VMEM((CHUNK,), jnp.int32),
            pltpu.VMEM((CHUNK,), jnp.float32),
            pltpu.VMEM((CHUNK,), jnp.float32),
            pltpu.SemaphoreType.DMA,
            pltpu.SemaphoreType.DMA,
            pltpu.SemaphoreType.DMA,
            pltpu.SemaphoreType.DMA,
        ],
    )(pair_i, pair_j, d2, atom_types, sig_flat, eps_flat)


def kernel(pair_i, pair_j, pair_diff, atom_types, sig, eps):
    d2 = _d2_tc(pair_diff)
    return _lj_sc(pair_i, pair_j, d2, atom_types,
                  sig.reshape(64), eps.reshape(64))
